# baseline (device time: 769101 ns/iter reference)
import jax
import jax.numpy as jnp
from jax import lax
from jax.experimental import pallas as pl
from jax.experimental.pallas import tpu as pltpu

N_DEV = 4
M, K, N = 4096, 4096, 8192
BM = M // N_DEV
N_PASS = 1
NH = N // (2 * N_PASS)
N_STEPS = 2 * (N_DEV - 1)
EP_ROWS = 128
RS_ROWS = 512


def _ar_relu_body(p_ref, scale_ref, out_ref, comm, local, outv,
                  send_sems, recv_sems, local_sems, init_sems, out_sems,
                  credit_sems):
    i = lax.axis_index("i")
    dirs = ((1, 0), (-1, N // 2))

    def blk(c):
        return (i + (c % N_DEV)) % N_DEV

    def rows(b):
        return pl.ds(b * BM, BM)

    barrier_sem = pltpu.get_barrier_semaphore()
    for c in (1, 3):
        pl.semaphore_signal(barrier_sem, inc=1, device_id=(blk(c),),
                            device_id_type=pl.DeviceIdType.MESH)
    pl.semaphore_wait(barrier_sem, 2)

    prev = [None, None]
    cur = [None, None]
    lpt = [[None, None], [None, None]]
    pending_out = [None, None]

    def issue_tile(g, d, cofs, s, r):
        b = blk(-d * (s + 1))
        cp = pltpu.make_async_copy(
            p_ref.at[pl.ds(b * BM + r * RS_ROWS, RS_ROWS),
                     pl.ds(cofs, NH)],
            local.at[g, r % 2], local_sems.at[g, r % 2])
        cp.start()
        lpt[g][r % 2] = cp

    def epilogue(g, cofs, slot, b):
        for r in range(BM // EP_ROWS):
            if pending_out[g] is not None:
                pending_out[g].wait()
            outv[g] = jnp.maximum(
                comm[g, slot, pl.ds(r * EP_ROWS, EP_ROWS)].astype(
                    jnp.float32) * scale_ref[0, 0], 0.0)
            cp = pltpu.make_async_copy(
                outv.at[g],
                out_ref.at[pl.ds(b * BM + r * EP_ROWS, EP_ROWS),
                           pl.ds(cofs, NH)],
                out_sems.at[g])
            cp.start()
            pending_out[g] = cp

    for p in range(N_PASS):
        for s in range(N_STEPS):
            first = p == 0 and s == 0
            for g, (d, cbase) in enumerate(dirs):
                cofs = cbase + p * NH
                if not first:
                    prev[g].wait_send()
                    pl.semaphore_signal(credit_sems.at[g], inc=1,
                                        device_id=(blk(-d),),
                                        device_id_type=pl.DeviceIdType.MESH)
                    pl.semaphore_wait(credit_sems.at[g], 1)
                if s == 0:
                    ic = pltpu.make_async_copy(
                        p_ref.at[rows(blk(0)), pl.ds(cofs, NH)],
                        comm.at[g, 0], init_sems.at[g])
                    ic.start()
                    issue_tile(g, d, cofs, 0, 0)
                    issue_tile(g, d, cofs, 0, 1)
                    ic.wait()
                rdma = pltpu.make_async_remote_copy(
                    src_ref=comm.at[g, s % 2],
                    dst_ref=comm.at[g, (s + 1) % 2],
                    send_sem=send_sems.at[g],
                    recv_sem=recv_sems.at[g],
                    device_id=(blk(d),),
                    device_id_type=pl.DeviceIdType.MESH)
                rdma.start()
                cur[g] = rdma

            for g, (d, cbase) in enumerate(dirs):
                cofs = cbase + p * NH
                if 1 <= s <= N_DEV - 2:
                    issue_tile(g, d, cofs, s, 0)
                    issue_tile(g, d, cofs, s, 1)
                if s == N_DEV - 1:
                    epilogue(g, cofs, 1, blk(d))
                elif s == N_DEV:
                    epilogue(g, cofs, 0, blk(0))
                elif s == N_DEV + 1:
                    epilogue(g, cofs, 1, blk(-d))

            for g, (d, cbase) in enumerate(dirs):
                cur[g].wait_recv()
                if s <= N_DEV - 2:
                    for r in range(BM // RS_ROWS):
                        rs = pl.ds(r * RS_ROWS, RS_ROWS)
                        lpt[g][r % 2].wait()
                        comm[g, (s + 1) % 2, rs] = (
                            comm[g, (s + 1) % 2, rs] + local[g, r % 2])
                        if r + 2 < BM // RS_ROWS:
                            issue_tile(g, d, cofs, s, r + 2)
                prev[g] = cur[g]

        for g, (d, cbase) in enumerate(dirs):
            epilogue(g, cbase + p * NH, 0, blk(-2 * d))

    for g in range(2):
        prev[g].wait_send()
        pending_out[g].wait()


def kernel(x, w_mat, scale_x, scale_w):
    pbf = lax.dot_general(
        x, w_mat, (((1,), (0,)), ((), ())),
        preferred_element_type=jnp.bfloat16)
    s2 = (scale_x[0] * scale_w[0]).astype(jnp.float32).reshape(1, 1)

    return pl.pallas_call(
        _ar_relu_body,
        out_shape=jax.ShapeDtypeStruct((M, N), jnp.float32),
        in_specs=[pl.BlockSpec(memory_space=pl.ANY),
                  pl.BlockSpec(memory_space=pltpu.MemorySpace.SMEM)],
        out_specs=pl.BlockSpec(memory_space=pl.ANY),
        scratch_shapes=[
            pltpu.VMEM((2, 2, BM, NH), jnp.bfloat16),
            pltpu.VMEM((2, 2, RS_ROWS, NH), jnp.bfloat16),
            pltpu.VMEM((2, EP_ROWS, NH), jnp.float32),
            pltpu.SemaphoreType.DMA((2,)),
            pltpu.SemaphoreType.DMA((2,)),
            pltpu.SemaphoreType.DMA((2, 2)),
            pltpu.SemaphoreType.DMA((2,)),
            pltpu.SemaphoreType.DMA((2,)),
            pltpu.SemaphoreType.REGULAR((2,)),
        ],
        compiler_params=pltpu.CompilerParams(
            collective_id=0, vmem_limit_bytes=62 * 1024 * 1024),
    )(pbf, s2)


# device time: 727292 ns/iter; 1.0575x vs baseline; 1.0575x over previous
import jax
import jax.numpy as jnp
from jax import lax
from jax.experimental import pallas as pl
from jax.experimental.pallas import tpu as pltpu

N_DEV = 4
M, K, N = 4096, 4096, 8192
BM = M // N_DEV
N_PASS = 2
NH = N // (2 * N_PASS)
N_STEPS = 2 * (N_DEV - 1)
EP_ROWS = 256
MX_ROWS = 512


def _fused_body(x_ref, w_ref, scale_ref, out_ref, comm, local, outv,
                send_sems, recv_sems, out_sems, credit_sems):
    i = lax.axis_index("i")
    dirs = ((1, 0), (-1, N // 2))

    def blk(c):
        return (i + (c % N_DEV)) % N_DEV

    barrier_sem = pltpu.get_barrier_semaphore()
    for c in (1, 3):
        pl.semaphore_signal(barrier_sem, inc=1, device_id=(blk(c),),
                            device_id_type=pl.DeviceIdType.MESH)
    pl.semaphore_wait(barrier_sem, 2)

    prev = [None, None]
    cur = [None, None]
    pending_out = [None, None]

    def partial_tile(b, cofs, t):
        xs = x_ref[pl.ds(b * BM + t * MX_ROWS, MX_ROWS), :]
        ws = w_ref[:, pl.ds(cofs, NH)]
        return jax.lax.dot_general(
            xs, ws, (((1,), (0,)), ((), ())),
            preferred_element_type=jnp.float32).astype(jnp.bfloat16)

    def epilogue(g, cofs, slot, b):
        for r in range(BM // EP_ROWS):
            if pending_out[g] is not None:
                pending_out[g].wait()
            outv[g] = jnp.maximum(
                comm[g, slot, pl.ds(r * EP_ROWS, EP_ROWS)].astype(
                    jnp.float32) * scale_ref[0, 0], 0.0)
            cp = pltpu.make_async_copy(
                outv.at[g],
                out_ref.at[pl.ds(b * BM + r * EP_ROWS, EP_ROWS),
                           pl.ds(cofs, NH)],
                out_sems.at[g])
            cp.start()
            pending_out[g] = cp

    for p in range(N_PASS):
        for s in range(N_STEPS):
            first = p == 0 and s == 0
            for g, (d, cbase) in enumerate(dirs):
                cofs = cbase + p * NH
                if not first:
                    prev[g].wait_send()
                    pl.semaphore_signal(credit_sems.at[g], inc=1,
                                        device_id=(blk(-d),),
                                        device_id_type=pl.DeviceIdType.MESH)
                    pl.semaphore_wait(credit_sems.at[g], 1)
                if s == 0:
                    for t in range(BM // MX_ROWS):
                        comm[g, 0, pl.ds(t * MX_ROWS, MX_ROWS)] = (
                            partial_tile(blk(0), cofs, t))
                rdma = pltpu.make_async_remote_copy(
                    src_ref=comm.at[g, s % 2],
                    dst_ref=comm.at[g, (s + 1) % 2],
                    send_sem=send_sems.at[g],
                    recv_sem=recv_sems.at[g],
                    device_id=(blk(d),),
                    device_id_type=pl.DeviceIdType.MESH)
                rdma.start()
                cur[g] = rdma

            for g, (d, cbase) in enumerate(dirs):
                cofs = cbase + p * NH
                if s <= N_DEV - 2:
                    for t in range(BM // MX_ROWS):
                        local[g, pl.ds(t * MX_ROWS, MX_ROWS)] = (
                            partial_tile(blk(-d * (s + 1)), cofs, t))
                if s == N_DEV - 1:
                    epilogue(g, cofs, 1, blk(d))
                elif s == N_DEV:
                    epilogue(g, cofs, 0, blk(0))
                elif s == N_DEV + 1:
                    epilogue(g, cofs, 1, blk(-d))

            for g, (d, cbase) in enumerate(dirs):
                cur[g].wait_recv()
                if s <= N_DEV - 2:
                    for t in range(BM // MX_ROWS):
                        rs = pl.ds(t * MX_ROWS, MX_ROWS)
                        comm[g, (s + 1) % 2, rs] = (
                            comm[g, (s + 1) % 2, rs] + local[g, rs])
                prev[g] = cur[g]

        for g, (d, cbase) in enumerate(dirs):
            epilogue(g, cbase + p * NH, 0, blk(-2 * d))

    for g in range(2):
        prev[g].wait_send()
        pending_out[g].wait()


def kernel(x, w_mat, scale_x, scale_w):
    s2 = (scale_x[0] * scale_w[0]).astype(jnp.float32).reshape(1, 1)
    xq = x.astype(jnp.float8_e5m2)
    wq = w_mat.astype(jnp.float8_e5m2)
    return pl.pallas_call(
        _fused_body,
        out_shape=jax.ShapeDtypeStruct((M, N), jnp.float32),
        in_specs=[
            pl.BlockSpec(memory_space=pltpu.MemorySpace.VMEM),
            pl.BlockSpec(memory_space=pltpu.MemorySpace.VMEM),
            pl.BlockSpec(memory_space=pltpu.MemorySpace.SMEM),
        ],
        out_specs=pl.BlockSpec(memory_space=pl.ANY),
        scratch_shapes=[
            pltpu.VMEM((2, 2, BM, NH), jnp.bfloat16),
            pltpu.VMEM((2, BM, NH), jnp.bfloat16),
            pltpu.VMEM((2, EP_ROWS, NH), jnp.float32),
            pltpu.SemaphoreType.DMA((2,)),
            pltpu.SemaphoreType.DMA((2,)),
            pltpu.SemaphoreType.DMA((2,)),
            pltpu.SemaphoreType.REGULAR((2,)),
        ],
        compiler_params=pltpu.CompilerParams(
            collective_id=0, vmem_limit_bytes=62 * 1024 * 1024),
    )(xq, wq, s2)


# device time: 717121 ns/iter; 1.0725x vs baseline; 1.0142x over previous
import jax
import jax.numpy as jnp
from jax import lax
from jax.experimental import pallas as pl
from jax.experimental.pallas import tpu as pltpu

N_DEV = 4
M, K, N = 4096, 4096, 8192
BM = M // N_DEV
N_PASS = 2
NH = N // (2 * N_PASS)
N_STEPS = 2 * (N_DEV - 1)
EP_ROWS = 256
MX_ROWS = 512


def _fused_body(x_ref, w_ref, scale_ref, out_ref, comm, local, outv,
                send_sems, recv_sems, out_sems, credit_sems):
    i = lax.axis_index("i")
    dirs = ((1, 0), (-1, N // 2))

    def blk(c):
        return (i + (c % N_DEV)) % N_DEV

    barrier_sem = pltpu.get_barrier_semaphore()
    for c in (1, 3):
        pl.semaphore_signal(barrier_sem, inc=1, device_id=(blk(c),),
                            device_id_type=pl.DeviceIdType.MESH)
    pl.semaphore_wait(barrier_sem, 2)

    prev = [None, None]
    cur = [None, None]
    pending_out = [None, None]

    def partial_tile(b, cofs, t):
        xs = x_ref[pl.ds(b * BM + t * MX_ROWS, MX_ROWS), :]
        ws = w_ref[:, pl.ds(cofs, NH)]
        return jax.lax.dot_general(
            xs, ws, (((1,), (0,)), ((), ())),
            preferred_element_type=jnp.float32).astype(jnp.bfloat16)

    def epilogue(g, cofs, slot, b):
        for r in range(BM // EP_ROWS):
            if pending_out[g] is not None:
                pending_out[g].wait()
            outv[g] = jnp.maximum(
                comm[g, slot, pl.ds(r * EP_ROWS, EP_ROWS)].astype(
                    jnp.float32) * scale_ref[0, 0], 0.0)
            cp = pltpu.make_async_copy(
                outv.at[g],
                out_ref.at[pl.ds(b * BM + r * EP_ROWS, EP_ROWS),
                           pl.ds(cofs, NH)],
                out_sems.at[g])
            cp.start()
            pending_out[g] = cp

    for p in range(N_PASS):
        for s in range(N_STEPS):
            first = p == 0 and s == 0
            for g, (d, cbase) in enumerate(dirs):
                cofs = cbase + p * NH
                if not first:
                    pl.semaphore_wait(credit_sems.at[g], 1)
                if s == 0:
                    for t in range(BM // MX_ROWS):
                        comm[g, 0, pl.ds(t * MX_ROWS, MX_ROWS)] = (
                            partial_tile(blk(0), cofs, t))
                rdma = pltpu.make_async_remote_copy(
                    src_ref=comm.at[g, s % 2],
                    dst_ref=comm.at[g, (s + 1) % 2],
                    send_sem=send_sems.at[g],
                    recv_sem=recv_sems.at[g],
                    device_id=(blk(d),),
                    device_id_type=pl.DeviceIdType.MESH)
                rdma.start()
                cur[g] = rdma

            for g, (d, cbase) in enumerate(dirs):
                cofs = cbase + p * NH
                if s <= N_DEV - 2:
                    for t in range(BM // MX_ROWS):
                        local[g, pl.ds(t * MX_ROWS, MX_ROWS)] = (
                            partial_tile(blk(-d * (s + 1)), cofs, t))
                if s == N_DEV - 1:
                    epilogue(g, cofs, 1, blk(d))
                elif s == N_DEV:
                    epilogue(g, cofs, 0, blk(0))
                elif s == N_DEV + 1:
                    epilogue(g, cofs, 1, blk(-d))

            for g, (d, cbase) in enumerate(dirs):
                cur[g].wait_recv()
                cur[g].wait_send()
                if not (p == N_PASS - 1 and s == N_STEPS - 1):
                    pl.semaphore_signal(credit_sems.at[g], inc=1,
                                        device_id=(blk(-d),),
                                        device_id_type=pl.DeviceIdType.MESH)
                if s <= N_DEV - 2:
                    for t in range(BM // MX_ROWS):
                        rs = pl.ds(t * MX_ROWS, MX_ROWS)
                        comm[g, (s + 1) % 2, rs] = (
                            comm[g, (s + 1) % 2, rs] + local[g, rs])
                prev[g] = cur[g]

        for g, (d, cbase) in enumerate(dirs):
            epilogue(g, cbase + p * NH, 0, blk(-2 * d))

    for g in range(2):
        pending_out[g].wait()


def kernel(x, w_mat, scale_x, scale_w):
    s2 = (scale_x[0] * scale_w[0]).astype(jnp.float32).reshape(1, 1)
    xq = x.astype(jnp.float8_e5m2)
    wq = w_mat.astype(jnp.float8_e5m2)
    return pl.pallas_call(
        _fused_body,
        out_shape=jax.ShapeDtypeStruct((M, N), jnp.float32),
        in_specs=[
            pl.BlockSpec(memory_space=pltpu.MemorySpace.VMEM),
            pl.BlockSpec(memory_space=pltpu.MemorySpace.VMEM),
            pl.BlockSpec(memory_space=pltpu.MemorySpace.SMEM),
        ],
        out_specs=pl.BlockSpec(memory_space=pl.ANY),
        scratch_shapes=[
            pltpu.VMEM((2, 2, BM, NH), jnp.bfloat16),
            pltpu.VMEM((2, BM, NH), jnp.bfloat16),
            pltpu.VMEM((2, EP_ROWS, NH), jnp.float32),
            pltpu.SemaphoreType.DMA((2,)),
            pltpu.SemaphoreType.DMA((2,)),
            pltpu.SemaphoreType.DMA((2,)),
            pltpu.SemaphoreType.REGULAR((2,)),
        ],
        compiler_params=pltpu.CompilerParams(
            collective_id=0, vmem_limit_bytes=62 * 1024 * 1024),
    )(xq, wq, s2)
